# Initial kernel scaffold; baseline (speedup 1.0000x reference)
import jax
import jax.numpy as jnp
from jax.experimental import pallas as pl
from jax.experimental.pallas import tpu as pltpu

Q = 1024
D = 128
KB = 1024
NPAD = 100352
NBLK = NPAD // KB
TOPK = 10
SW = 128
BIGI = jnp.int32(2**30)


def _body(x_ref, ksq_ref, qsq_ref, kt_ref, outd_ref, outi_ref,
          sv_ref, si_ref):
    j = pl.program_id(0)

    @pl.when(j == 0)
    def _init():
        sv_ref[...] = jnp.full((Q, SW), jnp.inf, jnp.float32)
        si_ref[...] = jnp.zeros((Q, SW), jnp.int32)

    dot = jnp.dot(x_ref[...], kt_ref[...],
                  preferred_element_type=jnp.float32)
    dists = qsq_ref[...] - 2.0 * dot + ksq_ref[...]

    c = jnp.concatenate([sv_ref[...], dists], axis=1)
    kidx = jax.lax.broadcasted_iota(jnp.int32, (Q, KB), 1) + j * KB
    ci = jnp.concatenate([si_ref[...], kidx], axis=1)
    lane = jax.lax.broadcasted_iota(jnp.int32, (Q, SW + KB), 1)

    vals, idxs = [], []
    for _ in range(TOPK):
        m = jnp.min(c, axis=1, keepdims=True)
        is_min = c == m
        pos = jnp.where(is_min, lane, BIGI)
        p = jnp.min(pos, axis=1, keepdims=True)
        sel = lane == p
        iv = jnp.min(jnp.where(sel, ci, BIGI), axis=1, keepdims=True)
        vals.append(m)
        idxs.append(iv)
        c = jnp.where(sel, jnp.inf, c)

    pad_v = jnp.full((Q, SW - TOPK), jnp.inf, jnp.float32)
    pad_i = jnp.zeros((Q, SW - TOPK), jnp.int32)
    sv_ref[...] = jnp.concatenate(vals + [pad_v], axis=1)
    si_ref[...] = jnp.concatenate(idxs + [pad_i], axis=1)

    @pl.when(j == NBLK - 1)
    def _final():
        outd_ref[...] = jnp.concatenate(vals, axis=1)
        outi_ref[...] = jnp.concatenate(idxs, axis=1)


def kernel(x, keys, k):
    n = keys.shape[0]
    q_sq = jnp.sum(x * x, axis=1, keepdims=True)
    k_sq = jnp.sum(keys * keys, axis=1)[None, :]
    k_sq = jnp.pad(k_sq, ((0, 0), (0, NPAD - n)), constant_values=1e30)
    keys_t = jnp.pad(keys, ((0, NPAD - n), (0, 0))).T

    grid_spec = pltpu.PrefetchScalarGridSpec(
        num_scalar_prefetch=0,
        grid=(NBLK,),
        in_specs=[
            pl.BlockSpec((Q, D), lambda j: (0, 0)),
            pl.BlockSpec((1, KB), lambda j: (0, j)),
            pl.BlockSpec((Q, 1), lambda j: (0, 0)),
            pl.BlockSpec((D, KB), lambda j: (0, j)),
        ],
        out_specs=[
            pl.BlockSpec((Q, TOPK), lambda j: (0, 0)),
            pl.BlockSpec((Q, TOPK), lambda j: (0, 0)),
        ],
        scratch_shapes=[
            pltpu.VMEM((Q, SW), jnp.float32),
            pltpu.VMEM((Q, SW), jnp.int32),
        ],
    )
    outd, outi = pl.pallas_call(
        _body,
        grid_spec=grid_spec,
        out_shape=[
            jax.ShapeDtypeStruct((Q, TOPK), jnp.float32),
            jax.ShapeDtypeStruct((Q, TOPK), jnp.int32),
        ],
        compiler_params=pltpu.CompilerParams(
            dimension_semantics=("arbitrary",),
        ),
    )(x, k_sq, q_sq, keys_t)
    return (outd, outi)


# speedup vs baseline: 1.4062x; 1.4062x over previous
import jax
import jax.numpy as jnp
from jax.experimental import pallas as pl
from jax.experimental.pallas import tpu as pltpu

Q = 1024
D = 128
KB = 1024
NPAD = 100352
NBLK = NPAD // KB
TOPK = 10
SW = 128
BIGI = 2**30


def _body(x_ref, ksq_ref, qsq_ref, kt_ref, outd_ref, outi_ref,
          sv_ref, si_ref):
    j = pl.program_id(0)

    @pl.when(j == 0)
    def _init():
        sv_ref[...] = jnp.full((Q, SW), jnp.inf, jnp.float32)
        si_ref[...] = jnp.zeros((Q, SW), jnp.int32)

    dot = jnp.dot(x_ref[...], kt_ref[...],
                  preferred_element_type=jnp.float32)
    dists = qsq_ref[...] - 2.0 * dot + ksq_ref[...]

    c = jnp.concatenate([sv_ref[...], dists], axis=1)
    kidx = jax.lax.broadcasted_iota(jnp.int32, (Q, KB), 1) + j * KB
    ci = jnp.concatenate([si_ref[...], kidx], axis=1)
    lane = jax.lax.broadcasted_iota(jnp.int32, (Q, SW + KB), 1)

    vals, idxs = [], []
    for _ in range(TOPK):
        m = jnp.min(c, axis=1, keepdims=True)
        is_min = c == m
        pos = jnp.where(is_min, lane, BIGI)
        p = jnp.min(pos, axis=1, keepdims=True)
        sel = lane == p
        iv = jnp.min(jnp.where(sel, ci, BIGI), axis=1, keepdims=True)
        vals.append(m)
        idxs.append(iv)
        c = jnp.where(sel, jnp.inf, c)

    pad_v = jnp.full((Q, SW - TOPK), jnp.inf, jnp.float32)
    pad_i = jnp.zeros((Q, SW - TOPK), jnp.int32)
    sv_ref[...] = jnp.concatenate(vals + [pad_v], axis=1)
    si_ref[...] = jnp.concatenate(idxs + [pad_i], axis=1)

    @pl.when(j == NBLK - 1)
    def _final():
        outd_ref[...] = jnp.concatenate(vals, axis=1)
        outi_ref[...] = jnp.concatenate(idxs, axis=1)


def kernel(x, keys, k):
    n = keys.shape[0]
    q_sq = jnp.sum(x * x, axis=1, keepdims=True)
    k_sq = jnp.sum(keys * keys, axis=1)[None, :]
    k_sq = jnp.pad(k_sq, ((0, 0), (0, NPAD - n)), constant_values=1e30)
    keys_t = jnp.pad(keys, ((0, NPAD - n), (0, 0))).T

    grid_spec = pltpu.PrefetchScalarGridSpec(
        num_scalar_prefetch=0,
        grid=(NBLK,),
        in_specs=[
            pl.BlockSpec((Q, D), lambda j: (0, 0)),
            pl.BlockSpec((1, KB), lambda j: (0, j)),
            pl.BlockSpec((Q, 1), lambda j: (0, 0)),
            pl.BlockSpec((D, KB), lambda j: (0, j)),
        ],
        out_specs=[
            pl.BlockSpec((Q, TOPK), lambda j: (0, 0)),
            pl.BlockSpec((Q, TOPK), lambda j: (0, 0)),
        ],
        scratch_shapes=[
            pltpu.VMEM((Q, SW), jnp.float32),
            pltpu.VMEM((Q, SW), jnp.int32),
        ],
    )
    outd, outi = pl.pallas_call(
        _body,
        grid_spec=grid_spec,
        out_shape=[
            jax.ShapeDtypeStruct((Q, TOPK), jnp.float32),
            jax.ShapeDtypeStruct((Q, TOPK), jnp.int32),
        ],
        compiler_params=pltpu.CompilerParams(
            dimension_semantics=("arbitrary",),
        ),
    )(x, k_sq, q_sq, keys_t)
    return (outd, outi)
